# Initial kernel scaffold; baseline (speedup 1.0000x reference)
import jax
import jax.numpy as jnp
from jax.experimental import pallas as pl


def kernel(adj, input, coe, W0, b0, W1, b1, W2, b2, W3, b3, Wout, bout):
    raise NotImplementedError("write your pallas kernel here")


# speedup vs baseline: 1016.9022x; 1016.9022x over previous
import math

import jax
import jax.numpy as jnp
from jax.experimental import pallas as pl
from jax.experimental.pallas import tpu as pltpu

HIDDEN = 256
LAYERS = 4
HEAD = HIDDEN // LAYERS
K = 10
B = 16
NNODE = 128

_BINOM = [math.comb(K, j) / (2.0 ** K) for j in range(K + 1)]


def _mm(a, b):
    return jax.lax.dot_general(
        a, b, (((1,), (0,)), ((), ())),
        precision=jax.lax.Precision.HIGHEST,
        preferred_element_type=jnp.float32)


def _bern_kernel(adj_ref, x_ref, coe_ref,
                 w0_ref, b0_ref, w1_ref, b1_ref, w2_ref, b2_ref,
                 w3_ref, b3_ref, wout_ref, bout_ref, out_ref):
    A = adj_ref[0]
    mask = (A > 0).astype(jnp.float32)
    deg = jnp.sum(mask, axis=0)
    dinv = jnp.where(deg > 0,
                     1.0 / jnp.sqrt(jnp.maximum(deg, 1.0)),
                     0.0)
    M = dinv[:, None] * mask.T * dinv[None, :]

    temp = jnp.maximum(coe_ref[0], 0.0)
    a = [temp[j] * _BINOM[j] for j in range(K + 1)]

    x0 = x_ref[0]
    ws = [w0_ref, w1_ref, w2_ref, w3_ref]
    bs = [b0_ref, b1_ref, b2_ref, b3_ref]
    cache = [x0]
    outs = []
    for layer in range(LAYERS):
        xin = cache[0] if layer == 0 else jnp.concatenate(cache, axis=1)
        xp = _mm(xin, ws[layer][...]) + bs[layer][0][None, :]
        us = [xp]
        h = xp
        for _ in range(K):
            h = h + _mm(M, h)
            us.append(h)
        s = a[K] * us[0]
        for t in range(K - 1, -1, -1):
            s = s - _mm(M, s) + a[t] * us[K - t]
        h = jnp.maximum(s, 0.0)
        outs.append(h)
        cache.append(h)

    bern = jnp.concatenate(outs, axis=1) + x0
    out_ref[0] = _mm(bern, wout_ref[...]) + bout_ref[0][None, :]


def kernel(adj, input, coe, W0, b0, W1, b1, W2, b2, W3, b3, Wout, bout):
    coe2 = coe.reshape(1, K + 1)
    biases = [b.reshape(1, -1) for b in (b0, b1, b2, b3, bout)]

    def fixed(arr):
        nd = arr.ndim
        return pl.BlockSpec(arr.shape, lambda b: (0,) * nd)

    in_specs = [
        pl.BlockSpec((1, NNODE, NNODE), lambda b: (b, 0, 0)),
        pl.BlockSpec((1, NNODE, HIDDEN), lambda b: (b, 0, 0)),
        fixed(coe2),
        fixed(W0), fixed(biases[0]),
        fixed(W1), fixed(biases[1]),
        fixed(W2), fixed(biases[2]),
        fixed(W3), fixed(biases[3]),
        fixed(Wout), fixed(biases[4]),
    ]
    out = pl.pallas_call(
        _bern_kernel,
        grid=(B,),
        in_specs=in_specs,
        out_specs=pl.BlockSpec((1, NNODE, HIDDEN), lambda b: (b, 0, 0)),
        out_shape=jax.ShapeDtypeStruct((B, NNODE, HIDDEN), jnp.float32),
        compiler_params=pltpu.CompilerParams(
            dimension_semantics=("parallel",)),
    )(adj, input, coe2, W0, biases[0], W1, biases[1], W2, biases[2],
      W3, biases[3], Wout, biases[4])
    return out


# speedup vs baseline: 1759.6103x; 1.7304x over previous
import math

import jax
import jax.numpy as jnp
from jax.experimental import pallas as pl
from jax.experimental.pallas import tpu as pltpu

HIDDEN = 256
LAYERS = 4
HEAD = HIDDEN // LAYERS
K = 10
B = 16
NNODE = 128

_BINOM = [math.comb(K, j) / (2.0 ** K) for j in range(K + 1)]


def _poly_coeffs(j):
    c = [0] * (K + 1)
    for p in range(j + 1):
        for q in range(K - j + 1):
            c[p + q] += (-1) ** p * math.comb(j, p) * math.comb(K - j, q)
    return c


_T = [[float(_poly_coeffs(j)[i]) for j in range(K + 1)] for i in range(K + 1)]


def _mm(a, b):
    return jax.lax.dot_general(
        a, b, (((1,), (0,)), ((), ())),
        precision=jax.lax.Precision.HIGHEST,
        preferred_element_type=jnp.float32)


def _bern_kernel(adj_ref, x_ref, coe_ref,
                 w0_ref, b0_ref, w1_ref, b1_ref, w2_ref, b2_ref,
                 w3_ref, b3_ref, wout_ref, bout_ref, out_ref):
    A = adj_ref[0]
    mask = (A > 0).astype(jnp.float32)
    deg = jnp.sum(mask, axis=0)
    dinv = jnp.where(deg > 0,
                     1.0 / jnp.sqrt(jnp.maximum(deg, 1.0)),
                     0.0)
    M = dinv[:, None] * mask.T * dinv[None, :]

    temp = jnp.maximum(coe_ref[0], 0.0)
    c = []
    for i in range(K + 1):
        ci = None
        for j in range(K + 1):
            w = _T[i][j] * _BINOM[j]
            if w == 0.0:
                continue
            term = temp[j] * w
            ci = term if ci is None else ci + term
        c.append(ci if ci is not None else 0.0)

    x0 = x_ref[0]
    ws = [w0_ref, w1_ref, w2_ref, w3_ref]
    bs = [b0_ref, b1_ref, b2_ref, b3_ref]
    cache = [x0]
    outs = []
    for layer in range(LAYERS):
        xin = cache[0] if layer == 0 else jnp.concatenate(cache, axis=1)
        xp = _mm(xin, ws[layer][...]) + bs[layer][0][None, :]
        s = c[K] * xp
        for i in range(K - 1, -1, -1):
            s = _mm(M, s) + c[i] * xp
        h = jnp.maximum(s, 0.0)
        outs.append(h)
        cache.append(h)

    bern = jnp.concatenate(outs, axis=1) + x0
    out_ref[0] = _mm(bern, wout_ref[...]) + bout_ref[0][None, :]


def kernel(adj, input, coe, W0, b0, W1, b1, W2, b2, W3, b3, Wout, bout):
    coe2 = coe.reshape(1, K + 1)
    biases = [b.reshape(1, -1) for b in (b0, b1, b2, b3, bout)]

    def fixed(arr):
        nd = arr.ndim
        return pl.BlockSpec(arr.shape, lambda b: (0,) * nd)

    in_specs = [
        pl.BlockSpec((1, NNODE, NNODE), lambda b: (b, 0, 0)),
        pl.BlockSpec((1, NNODE, HIDDEN), lambda b: (b, 0, 0)),
        fixed(coe2),
        fixed(W0), fixed(biases[0]),
        fixed(W1), fixed(biases[1]),
        fixed(W2), fixed(biases[2]),
        fixed(W3), fixed(biases[3]),
        fixed(Wout), fixed(biases[4]),
    ]
    out = pl.pallas_call(
        _bern_kernel,
        grid=(B,),
        in_specs=in_specs,
        out_specs=pl.BlockSpec((1, NNODE, HIDDEN), lambda b: (b, 0, 0)),
        out_shape=jax.ShapeDtypeStruct((B, NNODE, HIDDEN), jnp.float32),
        compiler_params=pltpu.CompilerParams(
            dimension_semantics=("parallel",)),
    )(adj, input, coe2, W0, biases[0], W1, biases[1], W2, biases[2],
      W3, biases[3], Wout, biases[4])
    return out


# speedup vs baseline: 2932.5213x; 1.6666x over previous
import math

import jax
import jax.numpy as jnp
from jax.experimental import pallas as pl
from jax.experimental.pallas import tpu as pltpu

HIDDEN = 256
LAYERS = 4
HEAD = HIDDEN // LAYERS
K = 10
B = 16
NNODE = 128

_BINOM = [math.comb(K, j) / (2.0 ** K) for j in range(K + 1)]


def _poly_coeffs(j):
    c = [0] * (K + 1)
    for p in range(j + 1):
        for q in range(K - j + 1):
            c[p + q] += (-1) ** p * math.comb(j, p) * math.comb(K - j, q)
    return c


_T = [[float(_poly_coeffs(j)[i]) for j in range(K + 1)] for i in range(K + 1)]


def _mm(a, b):
    return jax.lax.dot_general(
        a, b, (((1,), (0,)), ((), ())),
        precision=jax.lax.Precision.DEFAULT,
        preferred_element_type=jnp.float32)


def _bern_kernel(adj_ref, x_ref, coe_ref,
                 w0_ref, b0_ref, w1_ref, b1_ref, w2_ref, b2_ref,
                 w3_ref, b3_ref, wout_ref, bout_ref, out_ref):
    A = adj_ref[0]
    mask = (A > 0).astype(jnp.float32)
    deg = jnp.sum(mask, axis=0)
    dinv = jnp.where(deg > 0,
                     1.0 / jnp.sqrt(jnp.maximum(deg, 1.0)),
                     0.0)
    M = dinv[:, None] * mask.T * dinv[None, :]

    temp = jnp.maximum(coe_ref[0], 0.0)
    c = []
    for i in range(K + 1):
        ci = None
        for j in range(K + 1):
            w = _T[i][j] * _BINOM[j]
            if w == 0.0:
                continue
            term = temp[j] * w
            ci = term if ci is None else ci + term
        c.append(ci if ci is not None else 0.0)

    x0 = x_ref[0]
    ws = [w0_ref, w1_ref, w2_ref, w3_ref]
    bs = [b0_ref, b1_ref, b2_ref, b3_ref]
    cache = [x0]
    outs = []
    for layer in range(LAYERS):
        xin = cache[0] if layer == 0 else jnp.concatenate(cache, axis=1)
        xp = _mm(xin, ws[layer][...]) + bs[layer][0][None, :]
        s = c[K] * xp
        for i in range(K - 1, -1, -1):
            s = _mm(M, s) + c[i] * xp
        h = jnp.maximum(s, 0.0)
        outs.append(h)
        cache.append(h)

    bern = jnp.concatenate(outs, axis=1) + x0
    out_ref[0] = _mm(bern, wout_ref[...]) + bout_ref[0][None, :]


def kernel(adj, input, coe, W0, b0, W1, b1, W2, b2, W3, b3, Wout, bout):
    coe2 = coe.reshape(1, K + 1)
    biases = [b.reshape(1, -1) for b in (b0, b1, b2, b3, bout)]

    def fixed(arr):
        nd = arr.ndim
        return pl.BlockSpec(arr.shape, lambda b: (0,) * nd)

    in_specs = [
        pl.BlockSpec((1, NNODE, NNODE), lambda b: (b, 0, 0)),
        pl.BlockSpec((1, NNODE, HIDDEN), lambda b: (b, 0, 0)),
        fixed(coe2),
        fixed(W0), fixed(biases[0]),
        fixed(W1), fixed(biases[1]),
        fixed(W2), fixed(biases[2]),
        fixed(W3), fixed(biases[3]),
        fixed(Wout), fixed(biases[4]),
    ]
    out = pl.pallas_call(
        _bern_kernel,
        grid=(B,),
        in_specs=in_specs,
        out_specs=pl.BlockSpec((1, NNODE, HIDDEN), lambda b: (b, 0, 0)),
        out_shape=jax.ShapeDtypeStruct((B, NNODE, HIDDEN), jnp.float32),
        compiler_params=pltpu.CompilerParams(
            dimension_semantics=("parallel",)),
    )(adj, input, coe2, W0, biases[0], W1, biases[1], W2, biases[2],
      W3, biases[3], Wout, biases[4])
    return out


# speedup vs baseline: 3047.1154x; 1.0391x over previous
import math

import jax
import jax.numpy as jnp
from jax.experimental import pallas as pl
from jax.experimental.pallas import tpu as pltpu

HIDDEN = 256
LAYERS = 4
HEAD = HIDDEN // LAYERS
K = 10
B = 16
NNODE = 128

_BINOM = [math.comb(K, j) / (2.0 ** K) for j in range(K + 1)]


def _poly_coeffs(j):
    c = [0] * (K + 1)
    for p in range(j + 1):
        for q in range(K - j + 1):
            c[p + q] += (-1) ** p * math.comb(j, p) * math.comb(K - j, q)
    return c


_T = [[float(_poly_coeffs(j)[i]) for j in range(K + 1)] for i in range(K + 1)]


def _mm(a, b):
    return jax.lax.dot_general(
        a, b, (((1,), (0,)), ((), ())),
        precision=jax.lax.Precision.DEFAULT,
        preferred_element_type=jnp.float32)


_BPP = 4


def _bern_kernel(adj_ref, x_ref, coe_ref,
                 w0_ref, b0_ref, w1_ref, b1_ref, w2_ref, b2_ref,
                 w3_ref, b3_ref, wout_ref, bout_ref, out_ref):
    temp = jnp.maximum(coe_ref[0], 0.0)
    c = []
    for i in range(K + 1):
        ci = None
        for j in range(K + 1):
            w = _T[i][j] * _BINOM[j]
            if w == 0.0:
                continue
            term = temp[j] * w
            ci = term if ci is None else ci + term
        c.append(ci if ci is not None else 0.0)

    ws = [w0_ref, w1_ref, w2_ref, w3_ref]
    bs = [b0_ref, b1_ref, b2_ref, b3_ref]
    for p in range(_BPP):
        A = adj_ref[p]
        mask = (A > 0).astype(jnp.float32)
        deg = jnp.sum(mask, axis=0)
        dinv = jnp.where(deg > 0,
                         1.0 / jnp.sqrt(jnp.maximum(deg, 1.0)),
                         0.0)
        M = dinv[:, None] * mask.T * dinv[None, :]

        x0 = x_ref[p]
        cache = [x0]
        outs = []
        for layer in range(LAYERS):
            xin = cache[0] if layer == 0 else jnp.concatenate(cache, axis=1)
            xp = _mm(xin, ws[layer][...]) + bs[layer][0][None, :]
            s = c[K] * xp
            for i in range(K - 1, -1, -1):
                s = _mm(M, s) + c[i] * xp
            h = jnp.maximum(s, 0.0)
            outs.append(h)
            cache.append(h)

        bern = jnp.concatenate(outs, axis=1) + x0
        out_ref[p] = _mm(bern, wout_ref[...]) + bout_ref[0][None, :]


def kernel(adj, input, coe, W0, b0, W1, b1, W2, b2, W3, b3, Wout, bout):
    coe2 = coe.reshape(1, K + 1)
    biases = [b.reshape(1, -1) for b in (b0, b1, b2, b3, bout)]

    def fixed(arr):
        nd = arr.ndim
        return pl.BlockSpec(arr.shape, lambda b: (0,) * nd)

    in_specs = [
        pl.BlockSpec((_BPP, NNODE, NNODE), lambda b: (b, 0, 0)),
        pl.BlockSpec((_BPP, NNODE, HIDDEN), lambda b: (b, 0, 0)),
        fixed(coe2),
        fixed(W0), fixed(biases[0]),
        fixed(W1), fixed(biases[1]),
        fixed(W2), fixed(biases[2]),
        fixed(W3), fixed(biases[3]),
        fixed(Wout), fixed(biases[4]),
    ]
    out = pl.pallas_call(
        _bern_kernel,
        grid=(B // _BPP,),
        in_specs=in_specs,
        out_specs=pl.BlockSpec((_BPP, NNODE, HIDDEN), lambda b: (b, 0, 0)),
        out_shape=jax.ShapeDtypeStruct((B, NNODE, HIDDEN), jnp.float32),
        compiler_params=pltpu.CompilerParams(
            dimension_semantics=("parallel",)),
    )(adj, input, coe2, W0, biases[0], W1, biases[1], W2, biases[2],
      W3, biases[3], Wout, biases[4])
    return out


# speedup vs baseline: 8761.6270x; 2.8754x over previous
import math

import jax
import jax.numpy as jnp
from jax.experimental import pallas as pl
from jax.experimental.pallas import tpu as pltpu

HIDDEN = 256
LAYERS = 4
HEAD = HIDDEN // LAYERS
K = 10
B = 16
NNODE = 128

_BINOM = [math.comb(K, j) / (2.0 ** K) for j in range(K + 1)]


def _poly_coeffs(j):
    c = [0] * (K + 1)
    for p in range(j + 1):
        for q in range(K - j + 1):
            c[p + q] += (-1) ** p * math.comb(j, p) * math.comb(K - j, q)
    return c


_T = [[float(_poly_coeffs(j)[i]) for j in range(K + 1)] for i in range(K + 1)]


def _mm(a, b):
    return jax.lax.dot_general(
        a, b, (((1,), (0,)), ((), ())),
        precision=jax.lax.Precision.DEFAULT,
        preferred_element_type=jnp.float32)


_BPP = 4


def _bern_kernel(adj_ref, x_ref, coe_ref,
                 w0_ref, b0_ref, w1_ref, b1_ref, w2_ref, b2_ref,
                 w3_ref, b3_ref, wout_ref, bout_ref, out_ref):
    temp = jnp.maximum(coe_ref[0], 0.0)
    c = []
    for i in range(K + 1):
        ci = None
        for j in range(K + 1):
            w = _T[i][j] * _BINOM[j]
            if w == 0.0:
                continue
            term = temp[j] * w
            ci = term if ci is None else ci + term
        c.append(ci if ci is not None else 0.0)

    ws = [w0_ref, w1_ref, w2_ref, w3_ref]
    bs = [b0_ref, b1_ref, b2_ref, b3_ref]

    Ms = []
    for p in range(_BPP):
        A = adj_ref[p]
        mask = (A > 0).astype(jnp.float32)
        deg = jnp.sum(mask, axis=0)
        dinv = jnp.where(deg > 0,
                         1.0 / jnp.sqrt(jnp.maximum(deg, 1.0)),
                         0.0)
        Ms.append(dinv[:, None] * mask.T * dinv[None, :])

    x0s = [x_ref[p] for p in range(_BPP)]
    caches = [[x0s[p]] for p in range(_BPP)]
    for layer in range(LAYERS):
        xps = []
        for p in range(_BPP):
            xin = (caches[p][0] if layer == 0
                   else jnp.concatenate(caches[p], axis=1))
            xps.append(_mm(xin, ws[layer][...]) + bs[layer][0][None, :])
        ss = [c[K] * xps[p] for p in range(_BPP)]
        for i in range(K - 1, -1, -1):
            for p in range(_BPP):
                ss[p] = _mm(Ms[p], ss[p]) + c[i] * xps[p]
        for p in range(_BPP):
            caches[p].append(jnp.maximum(ss[p], 0.0))

    for p in range(_BPP):
        bern = jnp.concatenate(caches[p][1:], axis=1) + x0s[p]
        out_ref[p] = _mm(bern, wout_ref[...]) + bout_ref[0][None, :]


def kernel(adj, input, coe, W0, b0, W1, b1, W2, b2, W3, b3, Wout, bout):
    coe2 = coe.reshape(1, K + 1)
    biases = [b.reshape(1, -1) for b in (b0, b1, b2, b3, bout)]

    def fixed(arr):
        nd = arr.ndim
        return pl.BlockSpec(arr.shape, lambda b: (0,) * nd)

    in_specs = [
        pl.BlockSpec((_BPP, NNODE, NNODE), lambda b: (b, 0, 0)),
        pl.BlockSpec((_BPP, NNODE, HIDDEN), lambda b: (b, 0, 0)),
        fixed(coe2),
        fixed(W0), fixed(biases[0]),
        fixed(W1), fixed(biases[1]),
        fixed(W2), fixed(biases[2]),
        fixed(W3), fixed(biases[3]),
        fixed(Wout), fixed(biases[4]),
    ]
    out = pl.pallas_call(
        _bern_kernel,
        grid=(B // _BPP,),
        in_specs=in_specs,
        out_specs=pl.BlockSpec((_BPP, NNODE, HIDDEN), lambda b: (b, 0, 0)),
        out_shape=jax.ShapeDtypeStruct((B, NNODE, HIDDEN), jnp.float32),
        compiler_params=pltpu.CompilerParams(
            dimension_semantics=("parallel",)),
    )(adj, input, coe2, W0, biases[0], W1, biases[1], W2, biases[2],
      W3, biases[3], Wout, biases[4])
    return out


# speedup vs baseline: 12424.3840x; 1.4180x over previous
import math

import jax
import jax.numpy as jnp
from jax.experimental import pallas as pl
from jax.experimental.pallas import tpu as pltpu

HIDDEN = 256
LAYERS = 4
HEAD = HIDDEN // LAYERS
K = 10
B = 16
NNODE = 128

_BINOM = [math.comb(K, j) / (2.0 ** K) for j in range(K + 1)]


def _poly_coeffs(j):
    c = [0] * (K + 1)
    for p in range(j + 1):
        for q in range(K - j + 1):
            c[p + q] += (-1) ** p * math.comb(j, p) * math.comb(K - j, q)
    return c


_T = [[float(_poly_coeffs(j)[i]) for j in range(K + 1)] for i in range(K + 1)]


def _mm(a, b):
    return jax.lax.dot_general(
        a, b, (((1,), (0,)), ((), ())),
        precision=jax.lax.Precision.DEFAULT,
        preferred_element_type=jnp.float32)


_BPP = 4


def _bern_kernel(adj_ref, x_ref, coe_ref,
                 w0_ref, b0_ref, w1_ref, b1_ref, w2_ref, b2_ref,
                 w3_ref, b3_ref, wout_ref, bout_ref, out_ref):
    temp = jnp.maximum(coe_ref[0], 0.0)
    c = []
    for i in range(K + 1):
        ci = None
        for j in range(K + 1):
            w = _T[i][j] * _BINOM[j]
            if w == 0.0:
                continue
            term = temp[j] * w
            ci = term if ci is None else ci + term
        c.append(ci if ci is not None else 0.0)

    ws = [w0_ref, w1_ref, w2_ref, w3_ref]
    bs = [b0_ref, b1_ref, b2_ref, b3_ref]

    Ms, M4s = [], []
    for p in range(_BPP):
        A = adj_ref[p]
        mask = (A > 0).astype(jnp.float32)
        deg = jnp.sum(mask, axis=0)
        dinv = jnp.where(deg > 0,
                         1.0 / jnp.sqrt(jnp.maximum(deg, 1.0)),
                         0.0)
        Ms.append(dinv[:, None] * mask.T * dinv[None, :])
    M2s = [_mm(Ms[p], Ms[p]) for p in range(_BPP)]
    M4s = [_mm(M2s[p], M2s[p]) for p in range(_BPP)]

    x0s = [x_ref[p] for p in range(_BPP)]
    caches = [[x0s[p]] for p in range(_BPP)]
    for layer in range(LAYERS):
        xps = []
        for p in range(_BPP):
            xin = (caches[p][0] if layer == 0
                   else jnp.concatenate(caches[p], axis=1))
            xps.append(_mm(xin, ws[layer][...]) + bs[layer][0][None, :])
        x1s = [_mm(Ms[p], xps[p]) for p in range(_BPP)]
        x2s = [_mm(Ms[p], x1s[p]) for p in range(_BPP)]
        x3s = [_mm(Ms[p], x2s[p]) for p in range(_BPP)]
        ss = []
        for p in range(_BPP):
            pw = [xps[p], x1s[p], x2s[p], x3s[p]]
            c2x = c[8] * pw[0] + c[9] * pw[1] + c[10] * pw[2]
            c1x = c[4] * pw[0] + c[5] * pw[1] + c[6] * pw[2] + c[7] * pw[3]
            c0x = c[0] * pw[0] + c[1] * pw[1] + c[2] * pw[2] + c[3] * pw[3]
            ss.append((c1x, c0x, c2x))
        vs = [_mm(M4s[p], ss[p][2]) + ss[p][0] for p in range(_BPP)]
        vs = [_mm(M4s[p], vs[p]) + ss[p][1] for p in range(_BPP)]
        for p in range(_BPP):
            caches[p].append(jnp.maximum(vs[p], 0.0))

    for p in range(_BPP):
        bern = jnp.concatenate(caches[p][1:], axis=1) + x0s[p]
        out_ref[p] = _mm(bern, wout_ref[...]) + bout_ref[0][None, :]


def kernel(adj, input, coe, W0, b0, W1, b1, W2, b2, W3, b3, Wout, bout):
    coe2 = coe.reshape(1, K + 1)
    biases = [b.reshape(1, -1) for b in (b0, b1, b2, b3, bout)]

    def fixed(arr):
        nd = arr.ndim
        return pl.BlockSpec(arr.shape, lambda b: (0,) * nd)

    in_specs = [
        pl.BlockSpec((_BPP, NNODE, NNODE), lambda b: (b, 0, 0)),
        pl.BlockSpec((_BPP, NNODE, HIDDEN), lambda b: (b, 0, 0)),
        fixed(coe2),
        fixed(W0), fixed(biases[0]),
        fixed(W1), fixed(biases[1]),
        fixed(W2), fixed(biases[2]),
        fixed(W3), fixed(biases[3]),
        fixed(Wout), fixed(biases[4]),
    ]
    out = pl.pallas_call(
        _bern_kernel,
        grid=(B // _BPP,),
        in_specs=in_specs,
        out_specs=pl.BlockSpec((_BPP, NNODE, HIDDEN), lambda b: (b, 0, 0)),
        out_shape=jax.ShapeDtypeStruct((B, NNODE, HIDDEN), jnp.float32),
        compiler_params=pltpu.CompilerParams(
            dimension_semantics=("parallel",)),
    )(adj, input, coe2, W0, biases[0], W1, biases[1], W2, biases[2],
      W3, biases[3], Wout, biases[4])
    return out


# speedup vs baseline: 15891.8587x; 1.2791x over previous
import math

import jax
import jax.numpy as jnp
from jax.experimental import pallas as pl
from jax.experimental.pallas import tpu as pltpu

HIDDEN = 256
LAYERS = 4
HEAD = HIDDEN // LAYERS
K = 10
B = 16
NNODE = 128

_BINOM = [math.comb(K, j) / (2.0 ** K) for j in range(K + 1)]


def _poly_coeffs(j):
    c = [0] * (K + 1)
    for p in range(j + 1):
        for q in range(K - j + 1):
            c[p + q] += (-1) ** p * math.comb(j, p) * math.comb(K - j, q)
    return c


_T = [[float(_poly_coeffs(j)[i]) for j in range(K + 1)] for i in range(K + 1)]


def _mm(a, b):
    return jax.lax.dot_general(
        a, b, (((1,), (0,)), ((), ())),
        precision=jax.lax.Precision.DEFAULT,
        preferred_element_type=jnp.float32)


_BPP = 8


def _bern_kernel(adj_ref, x_ref, coe_ref,
                 w0_ref, b0_ref, w1_ref, b1_ref, w2_ref, b2_ref,
                 w3_ref, b3_ref, wout_ref, bout_ref, out_ref):
    temp = jnp.maximum(coe_ref[0], 0.0)
    c = []
    for i in range(K + 1):
        ci = None
        for j in range(K + 1):
            w = _T[i][j] * _BINOM[j]
            if w == 0.0:
                continue
            term = temp[j] * w
            ci = term if ci is None else ci + term
        c.append(ci if ci is not None else 0.0)

    ws = [w0_ref, w1_ref, w2_ref, w3_ref]
    bs = [b0_ref, b1_ref, b2_ref, b3_ref]

    Ms, M4s = [], []
    for p in range(_BPP):
        A = adj_ref[p]
        mask = (A > 0).astype(jnp.float32)
        deg = jnp.sum(mask, axis=0)
        dinv = jnp.where(deg > 0,
                         1.0 / jnp.sqrt(jnp.maximum(deg, 1.0)),
                         0.0)
        Ms.append(dinv[:, None] * mask.T * dinv[None, :])
    M2s = [_mm(Ms[p], Ms[p]) for p in range(_BPP)]
    M4s = [_mm(M2s[p], M2s[p]) for p in range(_BPP)]

    x0s = [x_ref[p] for p in range(_BPP)]
    caches = [[x0s[p]] for p in range(_BPP)]
    for layer in range(LAYERS):
        xps = []
        for p in range(_BPP):
            xin = (caches[p][0] if layer == 0
                   else jnp.concatenate(caches[p], axis=1))
            xps.append(_mm(xin, ws[layer][...]) + bs[layer][0][None, :])
        x1s = [_mm(Ms[p], xps[p]) for p in range(_BPP)]
        x2s = [_mm(Ms[p], x1s[p]) for p in range(_BPP)]
        x3s = [_mm(Ms[p], x2s[p]) for p in range(_BPP)]
        ss = []
        for p in range(_BPP):
            pw = [xps[p], x1s[p], x2s[p], x3s[p]]
            c2x = c[8] * pw[0] + c[9] * pw[1] + c[10] * pw[2]
            c1x = c[4] * pw[0] + c[5] * pw[1] + c[6] * pw[2] + c[7] * pw[3]
            c0x = c[0] * pw[0] + c[1] * pw[1] + c[2] * pw[2] + c[3] * pw[3]
            ss.append((c1x, c0x, c2x))
        vs = [_mm(M4s[p], ss[p][2]) + ss[p][0] for p in range(_BPP)]
        vs = [_mm(M4s[p], vs[p]) + ss[p][1] for p in range(_BPP)]
        for p in range(_BPP):
            caches[p].append(jnp.maximum(vs[p], 0.0))

    for p in range(_BPP):
        bern = jnp.concatenate(caches[p][1:], axis=1) + x0s[p]
        out_ref[p] = _mm(bern, wout_ref[...]) + bout_ref[0][None, :]


def kernel(adj, input, coe, W0, b0, W1, b1, W2, b2, W3, b3, Wout, bout):
    coe2 = coe.reshape(1, K + 1)
    biases = [b.reshape(1, -1) for b in (b0, b1, b2, b3, bout)]

    def fixed(arr):
        nd = arr.ndim
        return pl.BlockSpec(arr.shape, lambda b: (0,) * nd)

    in_specs = [
        pl.BlockSpec((_BPP, NNODE, NNODE), lambda b: (b, 0, 0)),
        pl.BlockSpec((_BPP, NNODE, HIDDEN), lambda b: (b, 0, 0)),
        fixed(coe2),
        fixed(W0), fixed(biases[0]),
        fixed(W1), fixed(biases[1]),
        fixed(W2), fixed(biases[2]),
        fixed(W3), fixed(biases[3]),
        fixed(Wout), fixed(biases[4]),
    ]
    out = pl.pallas_call(
        _bern_kernel,
        grid=(B // _BPP,),
        in_specs=in_specs,
        out_specs=pl.BlockSpec((_BPP, NNODE, HIDDEN), lambda b: (b, 0, 0)),
        out_shape=jax.ShapeDtypeStruct((B, NNODE, HIDDEN), jnp.float32),
        compiler_params=pltpu.CompilerParams(
            dimension_semantics=("parallel",)),
    )(adj, input, coe2, W0, biases[0], W1, biases[1], W2, biases[2],
      W3, biases[3], Wout, biases[4])
    return out


# speedup vs baseline: 17342.9450x; 1.0913x over previous
import math

import jax
import jax.numpy as jnp
from jax.experimental import pallas as pl
from jax.experimental.pallas import tpu as pltpu

HIDDEN = 256
LAYERS = 4
HEAD = HIDDEN // LAYERS
K = 10
B = 16
NNODE = 128

_BINOM = [math.comb(K, j) / (2.0 ** K) for j in range(K + 1)]


def _poly_coeffs(j):
    c = [0] * (K + 1)
    for p in range(j + 1):
        for q in range(K - j + 1):
            c[p + q] += (-1) ** p * math.comb(j, p) * math.comb(K - j, q)
    return c


_T = [[float(_poly_coeffs(j)[i]) for j in range(K + 1)] for i in range(K + 1)]


def _mm(a, b):
    return jax.lax.dot_general(
        a, b, (((1,), (0,)), ((), ())),
        precision=jax.lax.Precision.DEFAULT,
        preferred_element_type=jnp.float32)


_BPP = 16


def _bern_kernel(adj_ref, x_ref, coe_ref,
                 w0_ref, b0_ref, w1_ref, b1_ref, w2_ref, b2_ref,
                 w3_ref, b3_ref, wout_ref, bout_ref, out_ref):
    temp = jnp.maximum(coe_ref[0], 0.0)
    c = []
    for i in range(K + 1):
        ci = None
        for j in range(K + 1):
            w = _T[i][j] * _BINOM[j]
            if w == 0.0:
                continue
            term = temp[j] * w
            ci = term if ci is None else ci + term
        c.append(ci if ci is not None else 0.0)

    ws = [w0_ref, w1_ref, w2_ref, w3_ref]
    bs = [b0_ref, b1_ref, b2_ref, b3_ref]

    Ms, M4s = [], []
    for p in range(_BPP):
        A = adj_ref[p]
        mask = (A > 0).astype(jnp.float32)
        deg = jnp.sum(mask, axis=0)
        dinv = jnp.where(deg > 0,
                         1.0 / jnp.sqrt(jnp.maximum(deg, 1.0)),
                         0.0)
        Ms.append(dinv[:, None] * mask.T * dinv[None, :])
    M2s = [_mm(Ms[p], Ms[p]) for p in range(_BPP)]
    M4s = [_mm(M2s[p], M2s[p]) for p in range(_BPP)]

    x0s = [x_ref[p] for p in range(_BPP)]
    caches = [[x0s[p]] for p in range(_BPP)]
    for layer in range(LAYERS):
        xps = []
        for p in range(_BPP):
            xin = (caches[p][0] if layer == 0
                   else jnp.concatenate(caches[p], axis=1))
            xps.append(_mm(xin, ws[layer][...]) + bs[layer][0][None, :])
        x1s = [_mm(Ms[p], xps[p]) for p in range(_BPP)]
        x2s = [_mm(Ms[p], x1s[p]) for p in range(_BPP)]
        x3s = [_mm(Ms[p], x2s[p]) for p in range(_BPP)]
        ss = []
        for p in range(_BPP):
            pw = [xps[p], x1s[p], x2s[p], x3s[p]]
            c2x = c[8] * pw[0] + c[9] * pw[1] + c[10] * pw[2]
            c1x = c[4] * pw[0] + c[5] * pw[1] + c[6] * pw[2] + c[7] * pw[3]
            c0x = c[0] * pw[0] + c[1] * pw[1] + c[2] * pw[2] + c[3] * pw[3]
            ss.append((c1x, c0x, c2x))
        vs = [_mm(M4s[p], ss[p][2]) + ss[p][0] for p in range(_BPP)]
        vs = [_mm(M4s[p], vs[p]) + ss[p][1] for p in range(_BPP)]
        for p in range(_BPP):
            caches[p].append(jnp.maximum(vs[p], 0.0))

    for p in range(_BPP):
        bern = jnp.concatenate(caches[p][1:], axis=1) + x0s[p]
        out_ref[p] = _mm(bern, wout_ref[...]) + bout_ref[0][None, :]


def kernel(adj, input, coe, W0, b0, W1, b1, W2, b2, W3, b3, Wout, bout):
    coe2 = coe.reshape(1, K + 1)
    biases = [b.reshape(1, -1) for b in (b0, b1, b2, b3, bout)]

    def fixed(arr):
        nd = arr.ndim
        return pl.BlockSpec(arr.shape, lambda b: (0,) * nd)

    in_specs = [
        pl.BlockSpec((_BPP, NNODE, NNODE), lambda b: (b, 0, 0)),
        pl.BlockSpec((_BPP, NNODE, HIDDEN), lambda b: (b, 0, 0)),
        fixed(coe2),
        fixed(W0), fixed(biases[0]),
        fixed(W1), fixed(biases[1]),
        fixed(W2), fixed(biases[2]),
        fixed(W3), fixed(biases[3]),
        fixed(Wout), fixed(biases[4]),
    ]
    out = pl.pallas_call(
        _bern_kernel,
        grid=(B // _BPP,),
        in_specs=in_specs,
        out_specs=pl.BlockSpec((_BPP, NNODE, HIDDEN), lambda b: (b, 0, 0)),
        out_shape=jax.ShapeDtypeStruct((B, NNODE, HIDDEN), jnp.float32),
        compiler_params=pltpu.CompilerParams(
            dimension_semantics=("parallel",)),
    )(adj, input, coe2, W0, biases[0], W1, biases[1], W2, biases[2],
      W3, biases[3], Wout, biases[4])
    return out
